# Initial kernel scaffold; baseline (speedup 1.0000x reference)
import jax
import jax.numpy as jnp
from jax.experimental import pallas as pl


def kernel(nodes, to_neighs, u2e, W1, b1, W2, b2, W3, b3):
    raise NotImplementedError("write your pallas kernel here")


# speedup vs baseline: 2.8490x; 2.8490x over previous
import functools

import jax
import jax.numpy as jnp
from jax import lax
from jax.experimental import pallas as pl
from jax.experimental.pallas import tpu as pltpu
from jax.experimental.pallas import tpu_sc as plsc

_NC, _NS = 2, 16
_NW = _NC * _NS
_CHUNK = 120
_NBUF = 2


@functools.lru_cache(maxsize=None)
def _make_sc_gather(V, D, B):
    b_per_w = B // _NW
    nchunks = b_per_w // _CHUNK
    mesh = plsc.VectorSubcoreMesh(core_axis_name="c", subcore_axis_name="s")

    @functools.partial(
        pl.kernel,
        out_type=jax.ShapeDtypeStruct((B, D), jnp.float32),
        mesh=mesh,
        scratch_types=[
            pltpu.VMEM((b_per_w,), jnp.int32),
            pltpu.VMEM((_NBUF, _CHUNK, D), jnp.float32),
            [pltpu.SemaphoreType.DMA] * _NBUF,
        ],
    )
    def sc_gather(table_hbm, idx_hbm, out_hbm, idx_v, buf_v, sems):
        wid = lax.axis_index("s") * _NC + lax.axis_index("c")
        base = wid * b_per_w
        pltpu.sync_copy(idx_hbm.at[pl.ds(base, b_per_w)], idx_v)

        def start(ci, b):
            pltpu.async_copy(
                table_hbm.at[idx_v.at[pl.ds(ci * _CHUNK, _CHUNK)]],
                buf_v.at[b], sems[b])

        def wait(b):
            pltpu.make_async_copy(
                table_hbm.at[idx_v.at[pl.ds(0, _CHUNK)]],
                buf_v.at[b], sems[b]).wait()

        for b in range(_NBUF):
            start(b, b)

        def body(j, carry):
            for b in range(_NBUF):
                ci = j * _NBUF + b
                wait(b)
                pltpu.sync_copy(
                    buf_v.at[b],
                    out_hbm.at[pl.ds(base + ci * _CHUNK, _CHUNK)])

                @pl.when(ci + _NBUF < nchunks)
                def _():
                    start(ci + _NBUF, b)
            return carry

        lax.fori_loop(0, nchunks // _NBUF, body, 0)

    return sc_gather


def _attention_body(e_ref, u_ref, w1_ref, b1_ref, w2_ref, b2_ref, w3_ref,
                    o_ref):
    bn, k, d = e_ref.shape
    e3 = e_ref[...]
    e2 = e3.reshape(bn * k, d)
    w1 = w1_ref[...]
    pn = jnp.dot(u_ref[...], w1[d:, :],
                 preferred_element_type=jnp.float32) + b1_ref[...]
    h = jnp.dot(e2, w1[:d, :], preferred_element_type=jnp.float32)
    h = h + jnp.broadcast_to(pn[:, None, :], (bn, k, d)).reshape(bn * k, d)
    h = jnp.maximum(h, 0.0)
    h = jnp.dot(h, w2_ref[...], preferred_element_type=jnp.float32)
    h = jnp.maximum(h + b2_ref[...], 0.0)
    s = jnp.dot(h, w3_ref[...], preferred_element_type=jnp.float32)
    s3 = s.reshape(bn, k, 1)
    p = jnp.exp(s3 - jnp.max(s3, axis=1, keepdims=True))
    att = p / jnp.sum(p, axis=1, keepdims=True)
    o_ref[...] = jnp.sum(att * e3, axis=1)


@functools.lru_cache(maxsize=None)
def _make_attention(N, K, D, BN):
    grid = (N // BN,)
    return pl.pallas_call(
        _attention_body,
        grid=grid,
        in_specs=[
            pl.BlockSpec((BN, K, D), lambda i: (i, 0, 0)),
            pl.BlockSpec((BN, D), lambda i: (i, 0)),
            pl.BlockSpec((2 * D, D), lambda i: (0, 0)),
            pl.BlockSpec((1, D), lambda i: (0, 0)),
            pl.BlockSpec((D, D), lambda i: (0, 0)),
            pl.BlockSpec((1, D), lambda i: (0, 0)),
            pl.BlockSpec((D, 1), lambda i: (0, 0)),
        ],
        out_specs=pl.BlockSpec((BN, D), lambda i: (i, 0)),
        out_shape=jax.ShapeDtypeStruct((N, D), jnp.float32),
        compiler_params=pltpu.CompilerParams(
            dimension_semantics=("parallel",)),
    )


def kernel(nodes, to_neighs, u2e, W1, b1, W2, b2, W3, b3):
    N, K = to_neighs.shape
    V, D = u2e.shape
    nidx = N * K + N
    unit = _NW * _CHUNK * _NBUF
    B = ((nidx + unit - 1) // unit) * unit
    all_idx = jnp.concatenate([
        to_neighs.reshape(-1),
        nodes,
        jnp.zeros((B - nidx,), jnp.int32),
    ])
    gathered = _make_sc_gather(V, D, B)(u2e, all_idx)
    e3 = gathered[:N * K].reshape(N, K, D)
    u_rep = gathered[N * K:N * K + N]
    bn = 200
    return _make_attention(N, K, D, bn)(
        e3, u_rep, W1, b1.reshape(1, D), W2, b2.reshape(1, D), W3)


# speedup vs baseline: 2.8499x; 1.0003x over previous
import functools

import jax
import jax.numpy as jnp
from jax import lax
from jax.experimental import pallas as pl
from jax.experimental.pallas import tpu as pltpu
from jax.experimental.pallas import tpu_sc as plsc

_NC, _NS = 2, 16
_NW = _NC * _NS
_CHUNK = 120
_NBUF = 2


@functools.lru_cache(maxsize=None)
def _make_sc_gather(V, D, B):
    b_per_w = B // _NW
    nchunks = b_per_w // _CHUNK
    mesh = plsc.VectorSubcoreMesh(core_axis_name="c", subcore_axis_name="s")

    @functools.partial(
        pl.kernel,
        out_type=jax.ShapeDtypeStruct((B, D), jnp.float32),
        mesh=mesh,
        scratch_types=[
            pltpu.VMEM((b_per_w,), jnp.int32),
            pltpu.VMEM((_NBUF, _CHUNK, D), jnp.float32),
            [pltpu.SemaphoreType.DMA] * _NBUF,
        ],
    )
    def sc_gather(table_hbm, idx_hbm, out_hbm, idx_v, buf_v, sems):
        wid = lax.axis_index("s") * _NC + lax.axis_index("c")
        base = wid * b_per_w
        pltpu.sync_copy(idx_hbm.at[pl.ds(base, b_per_w)], idx_v)

        def start(ci, b):
            pltpu.async_copy(
                table_hbm.at[idx_v.at[pl.ds(ci * _CHUNK, _CHUNK)]],
                buf_v.at[b], sems[b])

        def wait(b):
            pltpu.make_async_copy(
                table_hbm.at[idx_v.at[pl.ds(0, _CHUNK)]],
                buf_v.at[b], sems[b]).wait()

        for b in range(_NBUF):
            start(b, b)

        def body(j, carry):
            for b in range(_NBUF):
                ci = j * _NBUF + b
                wait(b)
                pltpu.sync_copy(
                    buf_v.at[b],
                    out_hbm.at[pl.ds(base + ci * _CHUNK, _CHUNK)])

                @pl.when(ci + _NBUF < nchunks)
                def _():
                    start(ci + _NBUF, b)
            return carry

        lax.fori_loop(0, nchunks // _NBUF, body, 0)

    return sc_gather


def _attention_body(e_ref, u_ref, w1_ref, b1_ref, w2_ref, b2_ref, w3_ref,
                    o_ref):
    bn, k, d = e_ref.shape
    e3 = e_ref[...]
    e2 = e3.reshape(bn * k, d).astype(jnp.bfloat16)
    w1 = w1_ref[...]
    pn = jnp.dot(u_ref[...], w1[d:, :],
                 preferred_element_type=jnp.float32) + b1_ref[...]
    h = jnp.dot(e2, w1[:d, :].astype(jnp.bfloat16),
                preferred_element_type=jnp.float32)
    h = h + jnp.broadcast_to(pn[:, None, :], (bn, k, d)).reshape(bn * k, d)
    h = jnp.maximum(h, 0.0).astype(jnp.bfloat16)
    h = jnp.dot(h, w2_ref[...].astype(jnp.bfloat16),
                preferred_element_type=jnp.float32)
    h = jnp.maximum(h + b2_ref[...], 0.0).astype(jnp.bfloat16)
    s = jnp.dot(h, w3_ref[...].astype(jnp.bfloat16),
                preferred_element_type=jnp.float32)
    s3 = s.reshape(bn, k, 1)
    p = jnp.exp(s3 - jnp.max(s3, axis=1, keepdims=True))
    att = p / jnp.sum(p, axis=1, keepdims=True)
    o_ref[...] = jnp.sum(att * e3, axis=1)


@functools.lru_cache(maxsize=None)
def _make_attention(N, K, D, BN):
    grid = (N // BN,)
    return pl.pallas_call(
        _attention_body,
        grid=grid,
        in_specs=[
            pl.BlockSpec((BN, K, D), lambda i: (i, 0, 0)),
            pl.BlockSpec((BN, D), lambda i: (i, 0)),
            pl.BlockSpec((2 * D, D), lambda i: (0, 0)),
            pl.BlockSpec((1, D), lambda i: (0, 0)),
            pl.BlockSpec((D, D), lambda i: (0, 0)),
            pl.BlockSpec((1, D), lambda i: (0, 0)),
            pl.BlockSpec((D, 1), lambda i: (0, 0)),
        ],
        out_specs=pl.BlockSpec((BN, D), lambda i: (i, 0)),
        out_shape=jax.ShapeDtypeStruct((N, D), jnp.float32),
        compiler_params=pltpu.CompilerParams(
            dimension_semantics=("parallel",)),
    )


def kernel(nodes, to_neighs, u2e, W1, b1, W2, b2, W3, b3):
    N, K = to_neighs.shape
    V, D = u2e.shape
    nidx = N * K + N
    unit = _NW * _CHUNK * _NBUF
    B = ((nidx + unit - 1) // unit) * unit
    all_idx = jnp.concatenate([
        to_neighs.reshape(-1),
        nodes,
        jnp.zeros((B - nidx,), jnp.int32),
    ])
    gathered = _make_sc_gather(V, D, B)(u2e, all_idx)
    e3 = gathered[:N * K].reshape(N, K, D)
    u_rep = gathered[N * K:N * K + N]
    bn = 200
    return _make_attention(N, K, D, bn)(
        e3, u_rep, W1, b1.reshape(1, D), W2, b2.reshape(1, D), W3)


# speedup vs baseline: 3.8447x; 1.3490x over previous
import functools

import jax
import jax.numpy as jnp
from jax import lax
from jax.experimental import pallas as pl
from jax.experimental.pallas import tpu as pltpu
from jax.experimental.pallas import tpu_sc as plsc

_NC, _NS = 2, 16
_NW = _NC * _NS
_CHUNK = 120
_NBUF = 2


@functools.lru_cache(maxsize=None)
def _make_sc_gather(V, D, B):
    b_per_w = B // _NW
    nchunks = b_per_w // _CHUNK
    mesh = plsc.VectorSubcoreMesh(core_axis_name="c", subcore_axis_name="s")

    @functools.partial(
        pl.kernel,
        out_type=jax.ShapeDtypeStruct((B, D), jnp.float32),
        mesh=mesh,
        scratch_types=[
            pltpu.VMEM((b_per_w,), jnp.int32),
            pltpu.VMEM((_NBUF, _CHUNK, D), jnp.float32),
            [pltpu.SemaphoreType.DMA] * _NBUF,
        ],
    )
    def sc_gather(table_hbm, idx_hbm, out_hbm, idx_v, buf_v, sems):
        wid = lax.axis_index("s") * _NC + lax.axis_index("c")
        base = wid * b_per_w
        pltpu.sync_copy(idx_hbm.at[pl.ds(base, b_per_w)], idx_v)

        def start(ci, b):
            pltpu.async_copy(
                table_hbm.at[idx_v.at[pl.ds(ci * _CHUNK, _CHUNK)]],
                buf_v.at[b], sems[b])

        def wait(b):
            pltpu.make_async_copy(
                table_hbm.at[idx_v.at[pl.ds(0, _CHUNK)]],
                buf_v.at[b], sems[b]).wait()

        for b in range(_NBUF):
            start(b, b)

        def body(j, carry):
            for b in range(_NBUF):
                ci = j * _NBUF + b
                wait(b)
                pltpu.sync_copy(
                    buf_v.at[b],
                    out_hbm.at[pl.ds(base + ci * _CHUNK, _CHUNK)])

                @pl.when(ci + _NBUF < nchunks)
                def _():
                    start(ci + _NBUF, b)
            return carry

        lax.fori_loop(0, nchunks // _NBUF, body, 0)

    return sc_gather


def _attention_body(e_ref, u_ref, w1_ref, b1_ref, w2_ref, b2_ref, w3_ref,
                    o_ref):
    bn, d = u_ref.shape
    k = e_ref.shape[0] // bn
    e3 = e_ref[...].reshape(bn, k, d)
    e2 = e_ref[...].astype(jnp.bfloat16)
    w1 = w1_ref[...]
    pn = jnp.dot(u_ref[...], w1[d:, :],
                 preferred_element_type=jnp.float32) + b1_ref[...]
    h = jnp.dot(e2, w1[:d, :].astype(jnp.bfloat16),
                preferred_element_type=jnp.float32)
    h = h + jnp.broadcast_to(pn[:, None, :], (bn, k, d)).reshape(bn * k, d)
    h = jnp.maximum(h, 0.0).astype(jnp.bfloat16)
    h = jnp.dot(h, w2_ref[...].astype(jnp.bfloat16),
                preferred_element_type=jnp.float32)
    h = jnp.maximum(h + b2_ref[...], 0.0).astype(jnp.bfloat16)
    s = jnp.dot(h, w3_ref[...].astype(jnp.bfloat16),
                preferred_element_type=jnp.float32)
    s3 = s.reshape(bn, k, 1)
    p = jnp.exp(s3 - jnp.max(s3, axis=1, keepdims=True))
    att = p / jnp.sum(p, axis=1, keepdims=True)
    o_ref[...] = jnp.sum(att * e3, axis=1)


@functools.lru_cache(maxsize=None)
def _make_attention(N, K, D, BN, B):
    grid = (N // BN,)
    ublk0 = N * K // BN
    return pl.pallas_call(
        _attention_body,
        grid=grid,
        in_specs=[
            pl.BlockSpec((BN * K, D), lambda i: (i, 0)),
            pl.BlockSpec((BN, D), lambda i: (i + ublk0, 0)),
            pl.BlockSpec((2 * D, D), lambda i: (0, 0)),
            pl.BlockSpec((1, D), lambda i: (0, 0)),
            pl.BlockSpec((D, D), lambda i: (0, 0)),
            pl.BlockSpec((1, D), lambda i: (0, 0)),
            pl.BlockSpec((D, 1), lambda i: (0, 0)),
        ],
        out_specs=pl.BlockSpec((BN, D), lambda i: (i, 0)),
        out_shape=jax.ShapeDtypeStruct((N, D), jnp.float32),
        compiler_params=pltpu.CompilerParams(
            dimension_semantics=("parallel",)),
    )


def kernel(nodes, to_neighs, u2e, W1, b1, W2, b2, W3, b3):
    N, K = to_neighs.shape
    V, D = u2e.shape
    nidx = N * K + N
    unit = _NW * _CHUNK * _NBUF
    B = ((nidx + unit - 1) // unit) * unit
    all_idx = jnp.concatenate([
        to_neighs.reshape(-1),
        nodes,
        jnp.zeros((B - nidx,), jnp.int32),
    ])
    gathered = _make_sc_gather(V, D, B)(u2e, all_idx)
    bn = 200
    return _make_attention(N, K, D, bn, B)(
        gathered, gathered, W1, b1.reshape(1, D), W2, b2.reshape(1, D), W3)
